# Initial kernel scaffold; baseline (speedup 1.0000x reference)
import jax
import jax.numpy as jnp
from jax.experimental import pallas as pl


def kernel(x, emb):
    raise NotImplementedError("write your pallas kernel here")


# speedup vs baseline: 1.7083x; 1.7083x over previous
import jax
import jax.numpy as jnp
from jax.experimental import pallas as pl


def kernel(x, emb):
    S, B, D = x.shape
    n_src = emb.shape[0]
    seg_rows = S // n_src
    block_rows = 256
    blocks_per_seg = seg_rows // block_rows

    def body(emb_ref, x_ref, o_ref):
        o_ref[...] = x_ref[...] + emb_ref[...]

    return pl.pallas_call(
        body,
        grid=(n_src, blocks_per_seg),
        in_specs=[
            pl.BlockSpec((1, 1, D), lambda i, j: (i, 0, 0)),
            pl.BlockSpec((block_rows, B, D), lambda i, j: (i * blocks_per_seg + j, 0, 0)),
        ],
        out_specs=pl.BlockSpec((block_rows, B, D), lambda i, j: (i * blocks_per_seg + j, 0, 0)),
        out_shape=jax.ShapeDtypeStruct(x.shape, x.dtype),
    )(emb[:, None, :], x)
